# baseline (speedup 1.0000x reference)
import jax
import jax.numpy as jnp
from jax import lax
from jax.experimental import pallas as pl
from jax.experimental.pallas import tpu as pltpu
from jax.experimental.pallas import tpu_sc as plsc

VOCAB = 1000000
MAX_LEN = 512
N_SEG = 3
HIDDEN = 64
HPAD = 128
B, L = 1024, 200
N_TOK = B * L

NC, NS, LANES = 2, 16, 16
NW = NC * NS

TPW = N_TOK // NW
C = 320
G = 128
NCH = TPW // C
PSROWS = N_SEG * MAX_LEN
ROWS_PER_TILE = MAX_LEN // NS

_mesh = plsc.VectorSubcoreMesh(core_axis_name="c", subcore_axis_name="s")
_params = pltpu.CompilerParams(needs_layout_passes=False,
                               use_tc_tiling_on_sc=True)


def _lookup_body(tok_hbm, pos_hbm, seg_hbm, t_hbm, p_hbm, s_hbm, out_hbm,
                 ps_sh, pbuf, psbuf, s_v, tpad, psrows,
                 tokidx, posidx, segidx, psidx, sem, gsem, psem):
    cid = lax.axis_index("c")
    sid = lax.axis_index("s")
    wid = sid * NC + cid
    base = wid * TPW

    prow0 = sid * ROWS_PER_TILE
    pltpu.sync_copy(p_hbm.at[pl.ds(prow0, ROWS_PER_TILE)], pbuf)
    pltpu.sync_copy(s_hbm, s_v)
    for s in range(N_SEG):
        def srow_body(r, carry):
            for c in range(HPAD // LANES):
                sl = pl.ds(c * LANES, LANES)
                psbuf[r, sl] = pbuf[r, sl] + s_v[s, sl]
            return carry
        lax.fori_loop(0, ROWS_PER_TILE, srow_body, 0)
        pltpu.sync_copy(psbuf, ps_sh.at[pl.ds(s * MAX_LEN + prow0,
                                              ROWS_PER_TILE)])
    plsc.subcore_barrier()

    def chunk_body(ch, carry):
        off = base + ch * C
        d_tok = pltpu.make_async_copy(tok_hbm.at[pl.ds(off, C)], tokidx, sem)
        d_pos = pltpu.make_async_copy(pos_hbm.at[pl.ds(off, C)], posidx, sem)
        d_seg = pltpu.make_async_copy(seg_hbm.at[pl.ds(off, C)], segidx, sem)
        d_tok.start()
        d_pos.start()
        d_seg.start()
        d_tok.wait()
        d_pos.wait()
        d_seg.wait()

        def psx_body(g, carry2):
            sl = pl.ds(g * LANES, LANES)
            psidx[sl] = (segidx[sl] << 9) + posidx[sl]
            return carry2
        lax.fori_loop(0, C // LANES, psx_body, 0, unroll=4)

        ds_ = []
        o = 0
        while o < C:
            n = min(G, C - o)
            sl = pl.ds(o, n)
            d = pltpu.make_async_copy(t_hbm.at[tokidx.at[sl]],
                                      tpad.at[sl], gsem)
            d.start()
            ds_.append(d)
            d = pltpu.make_async_copy(ps_sh.at[psidx.at[sl]],
                                      psrows.at[sl], psem)
            d.start()
            ds_.append(d)
            o += n
        for d in ds_:
            d.wait()

        def add_body(r, carry2):
            for c in range(HIDDEN // LANES):
                sl = pl.ds(c * LANES, LANES)
                plsc.addupdate(tpad.at[r, sl], psrows[r, sl])
            return carry2
        lax.fori_loop(0, C, add_body, 0, unroll=2)

        pltpu.sync_copy(tpad, out_hbm.at[pl.ds(off, C)])
        return carry

    lax.fori_loop(0, NCH, chunk_body, 0)


@jax.jit
def _bert_embed(tok, pos, seg, t128, p_pad, s_pad):
    lookup = pl.kernel(
        _lookup_body,
        out_type=jax.ShapeDtypeStruct((N_TOK, HPAD), jnp.float32),
        mesh=_mesh,
        scratch_types=[
            pltpu.VMEM_SHARED((PSROWS, HPAD), jnp.float32),
            pltpu.VMEM((ROWS_PER_TILE, HPAD), jnp.float32),
            pltpu.VMEM((ROWS_PER_TILE, HPAD), jnp.float32),
            pltpu.VMEM((N_SEG, HPAD), jnp.float32),
            pltpu.VMEM((C, HPAD), jnp.float32),
            pltpu.VMEM((C, HPAD), jnp.float32),
            pltpu.VMEM((C,), jnp.int32),
            pltpu.VMEM((C,), jnp.int32),
            pltpu.VMEM((C,), jnp.int32),
            pltpu.VMEM((C,), jnp.int32),
            pltpu.SemaphoreType.DMA,
            pltpu.SemaphoreType.DMA,
            pltpu.SemaphoreType.DMA,
        ],
        compiler_params=_params,
    )
    return lookup(tok, pos, seg, t128, p_pad, s_pad)


def kernel(input_batch, segment, position, t_table, p_table, s_table):
    tok = input_batch.reshape(N_TOK)
    pos = position.reshape(N_TOK)
    seg = segment.reshape(N_TOK)
    t_pad = jnp.pad(t_table, ((0, 0), (0, HPAD - HIDDEN)))
    p_pad = jnp.pad(p_table, ((0, 0), (0, HPAD - HIDDEN)))
    s_pad = jnp.pad(s_table, ((0, 0), (0, HPAD - HIDDEN)))
    out = _bert_embed(tok, pos, seg, t_pad, p_pad, s_pad)
    return out[:, :HIDDEN].reshape(B, L, HIDDEN)


# speedup vs baseline: 1.0064x; 1.0064x over previous
import jax
import jax.numpy as jnp
from jax import lax
from jax.experimental import pallas as pl
from jax.experimental.pallas import tpu as pltpu
from jax.experimental.pallas import tpu_sc as plsc

VOCAB = 1000000
MAX_LEN = 512
N_SEG = 3
HIDDEN = 64
HPAD = 128
B, L = 1024, 200
N_TOK = B * L

NC, NS, LANES = 2, 16, 16
NW = NC * NS

TPW = N_TOK // NW
C = 400
G = 128
NCH = TPW // C
PSROWS = N_SEG * MAX_LEN
ROWS_PER_TILE = MAX_LEN // NS

_mesh = plsc.VectorSubcoreMesh(core_axis_name="c", subcore_axis_name="s")
_params = pltpu.CompilerParams(needs_layout_passes=False,
                               use_tc_tiling_on_sc=True)


def _lookup_body(tok_hbm, pos_hbm, seg_hbm, t_hbm, p_hbm, s_hbm, out_hbm,
                 ps_sh, pbuf, psbuf, s_v, tpad, psrows,
                 tokidx, posidx, segidx, psidx, sem, gsem, psem):
    cid = lax.axis_index("c")
    sid = lax.axis_index("s")
    wid = sid * NC + cid
    base = wid * TPW

    prow0 = sid * ROWS_PER_TILE
    pltpu.sync_copy(p_hbm.at[pl.ds(prow0, ROWS_PER_TILE)], pbuf)
    pltpu.sync_copy(s_hbm, s_v)
    for s in range(N_SEG):
        def srow_body(r, carry):
            for c in range(HPAD // LANES):
                sl = pl.ds(c * LANES, LANES)
                psbuf[r, sl] = pbuf[r, sl] + s_v[s, sl]
            return carry
        lax.fori_loop(0, ROWS_PER_TILE, srow_body, 0)
        pltpu.sync_copy(psbuf, ps_sh.at[pl.ds(s * MAX_LEN + prow0,
                                              ROWS_PER_TILE)])
    plsc.subcore_barrier()

    def chunk_body(ch, carry):
        off = base + ch * C
        d_tok = pltpu.make_async_copy(tok_hbm.at[pl.ds(off, C)], tokidx, sem)
        d_pos = pltpu.make_async_copy(pos_hbm.at[pl.ds(off, C)], posidx, sem)
        d_seg = pltpu.make_async_copy(seg_hbm.at[pl.ds(off, C)], segidx, sem)
        d_tok.start()
        d_pos.start()
        d_seg.start()
        d_tok.wait()
        d_pos.wait()
        d_seg.wait()

        def psx_body(g, carry2):
            sl = pl.ds(g * LANES, LANES)
            psidx[sl] = (segidx[sl] << 9) + posidx[sl]
            return carry2
        lax.fori_loop(0, C // LANES, psx_body, 0, unroll=4)

        ds_ = []
        o = 0
        while o < C:
            n = min(G, C - o)
            sl = pl.ds(o, n)
            d = pltpu.make_async_copy(t_hbm.at[tokidx.at[sl]],
                                      tpad.at[sl], gsem)
            d.start()
            ds_.append(d)
            d = pltpu.make_async_copy(ps_sh.at[psidx.at[sl]],
                                      psrows.at[sl], psem)
            d.start()
            ds_.append(d)
            o += n
        for d in ds_:
            d.wait()

        def add_body(r, carry2):
            for c in range(HIDDEN // LANES):
                sl = pl.ds(c * LANES, LANES)
                plsc.addupdate(tpad.at[r, sl], psrows[r, sl])
            return carry2
        lax.fori_loop(0, C, add_body, 0, unroll=2)

        pltpu.sync_copy(tpad, out_hbm.at[pl.ds(off, C)])
        return carry

    lax.fori_loop(0, NCH, chunk_body, 0)


@jax.jit
def _bert_embed(tok, pos, seg, t128, p_pad, s_pad):
    lookup = pl.kernel(
        _lookup_body,
        out_type=jax.ShapeDtypeStruct((N_TOK, HPAD), jnp.float32),
        mesh=_mesh,
        scratch_types=[
            pltpu.VMEM_SHARED((PSROWS, HPAD), jnp.float32),
            pltpu.VMEM((ROWS_PER_TILE, HPAD), jnp.float32),
            pltpu.VMEM((ROWS_PER_TILE, HPAD), jnp.float32),
            pltpu.VMEM((N_SEG, HPAD), jnp.float32),
            pltpu.VMEM((C, HPAD), jnp.float32),
            pltpu.VMEM((C, HPAD), jnp.float32),
            pltpu.VMEM((C,), jnp.int32),
            pltpu.VMEM((C,), jnp.int32),
            pltpu.VMEM((C,), jnp.int32),
            pltpu.VMEM((C,), jnp.int32),
            pltpu.SemaphoreType.DMA,
            pltpu.SemaphoreType.DMA,
            pltpu.SemaphoreType.DMA,
        ],
        compiler_params=_params,
    )
    return lookup(tok, pos, seg, t128, p_pad, s_pad)


def kernel(input_batch, segment, position, t_table, p_table, s_table):
    tok = input_batch.reshape(N_TOK)
    pos = position.reshape(N_TOK)
    seg = segment.reshape(N_TOK)
    t_pad = jnp.pad(t_table, ((0, 0), (0, HPAD - HIDDEN)))
    p_pad = jnp.pad(p_table, ((0, 0), (0, HPAD - HIDDEN)))
    s_pad = jnp.pad(s_table, ((0, 0), (0, HPAD - HIDDEN)))
    out = _bert_embed(tok, pos, seg, t_pad, p_pad, s_pad)
    return out[:, :HIDDEN].reshape(B, L, HIDDEN)
